# baseline (speedup 1.0000x reference)
import functools

import jax
import jax.numpy as jnp
from jax import lax
from jax.experimental import pallas as pl
from jax.experimental.pallas import tpu as pltpu
from jax.experimental.pallas import tpu_sc as plsc

_LANES = 16


@functools.lru_cache(maxsize=None)
def _make_sc_pool(B, S, V, D, CH):
    info = plsc.get_sparse_core_info()
    NC, NS = info.num_cores, info.num_subcores
    NW = NC * NS
    assert B % NW == 0 and S % CH == 0 and D % _LANES == 0
    rpw = B // NW
    cpr = S // CH
    nvec = D // _LANES
    assert cpr == 2, "row body is written for 2 chunks/row"
    inv_s = 1.0 / float(S)

    mesh = plsc.VectorSubcoreMesh(core_axis_name="c", subcore_axis_name="s")

    @functools.partial(
        pl.kernel,
        out_type=jax.ShapeDtypeStruct((B, D), jnp.float32),
        mesh=mesh,
        scratch_types=[
            pltpu.VMEM((rpw * cpr, CH), jnp.int32),
            pltpu.VMEM((CH, D), jnp.float32),
            pltpu.VMEM((CH, D), jnp.float32),
            pltpu.VMEM((rpw, D), jnp.float32),
            pltpu.SemaphoreType.DMA,
            pltpu.SemaphoreType.DMA,
        ],
        compiler_params=pltpu.CompilerParams(use_tc_tiling_on_sc=False),
    )
    def sc_pool(x_hbm, table_hbm, out_hbm, idx_v, buf0, buf1, pooled_v,
                sem0, sem1):
        wid = lax.axis_index("s") * NC + lax.axis_index("c")
        base = wid * rpw
        pltpu.sync_copy(x_hbm.at[pl.ds(base * cpr, rpw * cpr)], idx_v)

        def row_body(r, _):
            c0 = pltpu.async_copy(table_hbm.at[idx_v.at[cpr * r]], buf0, sem0)
            c1 = pltpu.async_copy(table_hbm.at[idx_v.at[cpr * r + 1]], buf1,
                                  sem1)
            c0.wait()
            c1.wait()

            def acc_body(l, accs):
                return tuple(
                    accs[k]
                    + buf0[l, pl.ds(_LANES * k, _LANES)]
                    + buf1[l, pl.ds(_LANES * k, _LANES)]
                    for k in range(nvec)
                )

            accs = lax.fori_loop(
                0, CH, acc_body,
                tuple(jnp.zeros((_LANES,), jnp.float32) for _ in range(nvec)))
            for k in range(nvec):
                pooled_v[r, pl.ds(_LANES * k, _LANES)] = accs[k] * inv_s
            return 0

        lax.fori_loop(0, rpw, row_body, 0)
        pltpu.sync_copy(pooled_v, out_hbm.at[pl.ds(base, rpw)])

    return sc_pool


def _proj_body(p_ref, w_ref, b_ref, o_ref):
    o_ref[...] = (
        lax.dot_general(p_ref[...], w_ref[...], (((1,), (1,)), ((), ())),
                        preferred_element_type=jnp.float32)
        + b_ref[...]
    )


@functools.lru_cache(maxsize=None)
def _make_proj(B, D, C):
    return pl.pallas_call(
        _proj_body,
        out_shape=jax.ShapeDtypeStruct((B, C), jnp.float32),
    )


@jax.jit
def kernel(x, emb_table, W, b):
    B, S = x.shape
    V, D = emb_table.shape
    C = W.shape[0]
    CH = 100
    x2 = x.astype(jnp.int32).reshape(B * S // CH, CH)
    pooled = _make_sc_pool(B, S, V, D, CH)(x2, emb_table)
    return _make_proj(B, D, C)(pooled, W, b.reshape(1, C))
